# baseline (speedup 1.0000x reference)
import jax
import jax.numpy as jnp
from jax import lax
from jax.experimental import pallas as pl
from jax.experimental.pallas import tpu as pltpu

N_TOKENS = 8192
IN_DIM = 4096
NUM_EXPERTS = 64
K = 8
NOISE_EPS = 0.01
BLK = 256
NBLK = N_TOKENS // BLK
INV_SQRT2 = 0.7071067811865476


def _router_kernel(
    x_ref, wg_ref, wn_ref, z_ref, gates_ref, load_ref, clean_s, std_s, noisy_s
):
    i = pl.program_id(0)
    cur = lax.rem(i, 2)
    prev = lax.rem(i + 1, 2)

    @pl.when(i == 0)
    def _():
        clean_s[1] = jnp.zeros((BLK, NUM_EXPERTS), jnp.float32)
        std_s[1] = jnp.ones((BLK, NUM_EXPERTS), jnp.float32)
        noisy_s[1] = jnp.zeros((BLK, NUM_EXPERTS), jnp.float32)
        load_ref[...] = jnp.zeros((8, NUM_EXPERTS), jnp.float32)

    x = x_ref[...]
    clean = jnp.dot(x, wg_ref[...], preferred_element_type=jnp.float32)
    raw = jnp.dot(x, wn_ref[...], preferred_element_type=jnp.float32)
    std = jnp.logaddexp(raw, 0.0) + NOISE_EPS
    clean_s[cur] = clean
    std_s[cur] = std
    noisy_s[cur] = clean + z_ref[...] * std

    cleanp = clean_s[prev]
    stdp = std_s[prev]
    noisy = noisy_s[prev]

    work = noisy
    neg = jnp.float32(-jnp.inf)
    t8 = None
    t9 = None
    for it in range(K + 1):
        m = jnp.max(work, axis=-1, keepdims=True)
        if it == K - 1:
            t8 = m
        if it == K:
            t9 = m
        else:
            work = jnp.where(work == m, neg, work)

    mask = noisy > t9
    mx = jnp.max(noisy, axis=-1, keepdims=True)
    e = jnp.where(mask, jnp.exp(noisy - mx), 0.0)
    denom = jnp.sum(e, axis=-1, keepdims=True)
    gates_ref[...] = jnp.where(mask, (e / denom + 0.01) * (1.0 / 1.08), 0.0)

    kthresh = jnp.where(mask, t9, t8)
    arg = (cleanp - kthresh) / stdp * INV_SQRT2
    probs = 0.5 * (1.0 + lax.erf(arg))
    part = jnp.sum(probs.reshape(BLK // 8, 8, NUM_EXPERTS), axis=0)
    load_ref[...] += jnp.where(i > 0, part, 0.0)


@jax.jit
def _impl(x, wg_t, wn_t):
    z = jax.random.normal(
        jax.random.key(42), (N_TOKENS, NUM_EXPERTS), dtype=jnp.float32
    )
    gates, load8 = pl.pallas_call(
        _router_kernel,
        grid=(NBLK + 1,),
        in_specs=[
            pl.BlockSpec((BLK, IN_DIM), lambda i: (jnp.minimum(i, NBLK - 1), 0)),
            pl.BlockSpec((IN_DIM, NUM_EXPERTS), lambda i: (0, 0)),
            pl.BlockSpec((IN_DIM, NUM_EXPERTS), lambda i: (0, 0)),
            pl.BlockSpec((BLK, NUM_EXPERTS), lambda i: (jnp.minimum(i, NBLK - 1), 0)),
        ],
        out_specs=[
            pl.BlockSpec((BLK, NUM_EXPERTS), lambda i: (jnp.maximum(i - 1, 0), 0)),
            pl.BlockSpec((8, NUM_EXPERTS), lambda i: (0, 0)),
        ],
        out_shape=[
            jax.ShapeDtypeStruct((N_TOKENS, NUM_EXPERTS), jnp.float32),
            jax.ShapeDtypeStruct((8, NUM_EXPERTS), jnp.float32),
        ],
        scratch_shapes=[
            pltpu.VMEM((2, BLK, NUM_EXPERTS), jnp.float32),
            pltpu.VMEM((2, BLK, NUM_EXPERTS), jnp.float32),
            pltpu.VMEM((2, BLK, NUM_EXPERTS), jnp.float32),
        ],
    )(x, wg_t, wn_t, z)
    return gates, jnp.sum(load8, axis=0)


def kernel(x, W_g, W_noise):
    return _impl(x, W_g.T, W_noise.T)


# speedup vs baseline: 1.3413x; 1.3413x over previous
import jax
import jax.numpy as jnp
from jax import lax
from jax.experimental import pallas as pl
from jax.experimental.pallas import tpu as pltpu

N_TOKENS = 8192
IN_DIM = 4096
NUM_EXPERTS = 64
K = 8
NOISE_EPS = 0.01
BLK = 256
NBLK = N_TOKENS // BLK
INV_SQRT2 = 0.7071067811865476

_Z = jax.random.normal(
    jax.random.key(42), (N_TOKENS, NUM_EXPERTS), dtype=jnp.float32
)


def _router_kernel(x_ref, wg_ref, wn_ref, z_ref, gates_ref, load_ref):
    x = x_ref[...]
    clean = jnp.dot(x, wg_ref[...], preferred_element_type=jnp.float32)
    raw = jnp.dot(x, wn_ref[...], preferred_element_type=jnp.float32)
    std = jnp.logaddexp(raw, 0.0) + NOISE_EPS
    noisy = clean + z_ref[...] * std

    work = noisy
    neg = jnp.float32(-jnp.inf)
    t8 = None
    t9 = None
    for it in range(K + 1):
        m = jnp.max(work, axis=-1, keepdims=True)
        if it == K - 1:
            t8 = m
        if it == K:
            t9 = m
        else:
            work = jnp.where(work == m, neg, work)

    mask = noisy > t9
    mx = jnp.max(noisy, axis=-1, keepdims=True)
    e = jnp.where(mask, jnp.exp(noisy - mx), 0.0)
    denom = jnp.sum(e, axis=-1, keepdims=True)
    gates_ref[...] = jnp.where(mask, (e / denom + 0.01) * (1.0 / 1.08), 0.0)

    kthresh = jnp.where(mask, t9, t8)
    arg = (clean - kthresh) / std * INV_SQRT2
    probs = 0.5 * (1.0 + lax.erf(arg))
    part = jnp.sum(probs.reshape(BLK // 8, 8, NUM_EXPERTS), axis=0)
    load_ref[...] = part.reshape(1, 8, NUM_EXPERTS)


@jax.jit
def _impl(x, wg_t, wn_t, z):
    gates, load_parts = pl.pallas_call(
        _router_kernel,
        grid=(NBLK,),
        in_specs=[
            pl.BlockSpec((BLK, IN_DIM), lambda i: (i, 0)),
            pl.BlockSpec((IN_DIM, NUM_EXPERTS), lambda i: (0, 0)),
            pl.BlockSpec((IN_DIM, NUM_EXPERTS), lambda i: (0, 0)),
            pl.BlockSpec((BLK, NUM_EXPERTS), lambda i: (i, 0)),
        ],
        out_specs=[
            pl.BlockSpec((BLK, NUM_EXPERTS), lambda i: (i, 0)),
            pl.BlockSpec((1, 8, NUM_EXPERTS), lambda i: (i, 0, 0)),
        ],
        out_shape=[
            jax.ShapeDtypeStruct((N_TOKENS, NUM_EXPERTS), jnp.float32),
            jax.ShapeDtypeStruct((NBLK, 8, NUM_EXPERTS), jnp.float32),
        ],
        compiler_params=pltpu.CompilerParams(
            dimension_semantics=("parallel",),
        ),
    )(x, wg_t, wn_t, z)
    return gates, jnp.sum(load_parts, axis=(0, 1))


def kernel(x, W_g, W_noise):
    return _impl(x, W_g.T, W_noise.T, _Z)


# speedup vs baseline: 1.6186x; 1.2067x over previous
import jax
import jax.numpy as jnp
from jax import lax
from jax.experimental import pallas as pl
from jax.experimental.pallas import tpu as pltpu

N_TOKENS = 8192
IN_DIM = 4096
NUM_EXPERTS = 64
K = 8
NOISE_EPS = 0.01
BLK = 512
NBLK = N_TOKENS // BLK
INV_SQRT2 = 0.7071067811865476

_Z = jax.random.normal(
    jax.random.key(42), (N_TOKENS, NUM_EXPERTS), dtype=jnp.float32
)


def _router_kernel(x_ref, wg_ref, wn_ref, z_ref, gates_ref, load_ref):
    x = x_ref[...]
    clean = jnp.dot(x, wg_ref[...], preferred_element_type=jnp.float32)
    raw = jnp.dot(x, wn_ref[...], preferred_element_type=jnp.float32)
    std = jnp.logaddexp(raw, 0.0) + NOISE_EPS
    noisy = clean + z_ref[...] * std

    work = noisy
    neg = jnp.float32(-jnp.inf)
    t8 = None
    t9 = None
    for it in range(K + 1):
        m = jnp.max(work, axis=-1, keepdims=True)
        if it == K - 1:
            t8 = m
        if it == K:
            t9 = m
        else:
            work = jnp.where(work == m, neg, work)

    mask = noisy > t9
    mx = jnp.max(noisy, axis=-1, keepdims=True)
    e = jnp.where(mask, jnp.exp(noisy - mx), 0.0)
    denom = jnp.sum(e, axis=-1, keepdims=True)
    gates_ref[...] = jnp.where(mask, (e / denom + 0.01) * (1.0 / 1.08), 0.0)

    kthresh = jnp.where(mask, t9, t8)
    arg = (clean - kthresh) / std * INV_SQRT2
    probs = 0.5 * (1.0 + lax.erf(arg))
    part = jnp.sum(probs.reshape(BLK // 8, 8, NUM_EXPERTS), axis=0)
    load_ref[...] = part.reshape(1, 8, NUM_EXPERTS)


@jax.jit
def _impl(x, wg_t, wn_t, z):
    gates, load_parts = pl.pallas_call(
        _router_kernel,
        grid=(NBLK,),
        in_specs=[
            pl.BlockSpec((BLK, IN_DIM), lambda i: (i, 0)),
            pl.BlockSpec((IN_DIM, NUM_EXPERTS), lambda i: (0, 0)),
            pl.BlockSpec((IN_DIM, NUM_EXPERTS), lambda i: (0, 0)),
            pl.BlockSpec((BLK, NUM_EXPERTS), lambda i: (i, 0)),
        ],
        out_specs=[
            pl.BlockSpec((BLK, NUM_EXPERTS), lambda i: (i, 0)),
            pl.BlockSpec((1, 8, NUM_EXPERTS), lambda i: (i, 0, 0)),
        ],
        out_shape=[
            jax.ShapeDtypeStruct((N_TOKENS, NUM_EXPERTS), jnp.float32),
            jax.ShapeDtypeStruct((NBLK, 8, NUM_EXPERTS), jnp.float32),
        ],
        compiler_params=pltpu.CompilerParams(
            dimension_semantics=("parallel",),
        ),
    )(x, wg_t, wn_t, z)
    return gates, jnp.sum(load_parts, axis=(0, 1))


def kernel(x, W_g, W_noise):
    return _impl(x, W_g.T, W_noise.T, _Z)


# speedup vs baseline: 1.7014x; 1.0511x over previous
import jax
import jax.numpy as jnp
from jax import lax
from jax.experimental import pallas as pl
from jax.experimental.pallas import tpu as pltpu

N_TOKENS = 8192
IN_DIM = 4096
NUM_EXPERTS = 64
K = 8
NOISE_EPS = 0.01
BLK = 1024
NBLK = N_TOKENS // BLK
INV_SQRT2 = 0.7071067811865476

_Z = jax.random.normal(
    jax.random.key(42), (N_TOKENS, NUM_EXPERTS), dtype=jnp.float32
)


def _router_kernel(x_ref, wg_ref, wn_ref, z_ref, gates_ref, load_ref):
    x = x_ref[...]
    clean = jnp.dot(x, wg_ref[...], preferred_element_type=jnp.float32)
    raw = jnp.dot(x, wn_ref[...], preferred_element_type=jnp.float32)
    std = jnp.logaddexp(raw, 0.0) + NOISE_EPS
    noisy = clean + z_ref[...] * std

    work = noisy
    neg = jnp.float32(-jnp.inf)
    t8 = None
    t9 = None
    for it in range(K + 1):
        m = jnp.max(work, axis=-1, keepdims=True)
        if it == K - 1:
            t8 = m
        if it == K:
            t9 = m
        else:
            work = jnp.where(work == m, neg, work)

    mask = noisy > t9
    mx = jnp.max(noisy, axis=-1, keepdims=True)
    e = jnp.where(mask, jnp.exp(noisy - mx), 0.0)
    denom = jnp.sum(e, axis=-1, keepdims=True)
    gates_ref[...] = jnp.where(mask, (e / denom + 0.01) * (1.0 / 1.08), 0.0)

    kthresh = jnp.where(mask, t9, t8)
    arg = (clean - kthresh) / std * INV_SQRT2
    probs = 0.5 * (1.0 + lax.erf(arg))
    part = jnp.sum(probs.reshape(BLK // 8, 8, NUM_EXPERTS), axis=0)
    load_ref[...] = part.reshape(1, 8, NUM_EXPERTS)


@jax.jit
def _impl(x, wg_t, wn_t, z):
    gates, load_parts = pl.pallas_call(
        _router_kernel,
        grid=(NBLK,),
        in_specs=[
            pl.BlockSpec((BLK, IN_DIM), lambda i: (i, 0)),
            pl.BlockSpec((IN_DIM, NUM_EXPERTS), lambda i: (0, 0)),
            pl.BlockSpec((IN_DIM, NUM_EXPERTS), lambda i: (0, 0)),
            pl.BlockSpec((BLK, NUM_EXPERTS), lambda i: (i, 0)),
        ],
        out_specs=[
            pl.BlockSpec((BLK, NUM_EXPERTS), lambda i: (i, 0)),
            pl.BlockSpec((1, 8, NUM_EXPERTS), lambda i: (i, 0, 0)),
        ],
        out_shape=[
            jax.ShapeDtypeStruct((N_TOKENS, NUM_EXPERTS), jnp.float32),
            jax.ShapeDtypeStruct((NBLK, 8, NUM_EXPERTS), jnp.float32),
        ],
        compiler_params=pltpu.CompilerParams(
            dimension_semantics=("parallel",),
        ),
    )(x, wg_t, wn_t, z)
    return gates, jnp.sum(load_parts, axis=(0, 1))


def kernel(x, W_g, W_noise):
    return _impl(x, W_g.T, W_noise.T, _Z)


# speedup vs baseline: 1.7239x; 1.0132x over previous
import jax
import jax.numpy as jnp
from jax import lax
from jax.experimental import pallas as pl
from jax.experimental.pallas import tpu as pltpu

N_TOKENS = 8192
IN_DIM = 4096
NUM_EXPERTS = 64
K = 8
NOISE_EPS = 0.01
BLK = 1024
NBLK = N_TOKENS // BLK
INV_SQRT2 = 0.7071067811865476

_Z = jax.random.normal(
    jax.random.key(42), (N_TOKENS, NUM_EXPERTS), dtype=jnp.float32
)


def _router_kernel(
    x_ref, wg_ref, wn_ref, z_ref, gates_ref, load_ref,
    c0, s0, n0, c1, s1, n1,
):
    i = pl.program_id(0)

    def phase_a(cs, ss, ns):
        x = x_ref[...]
        clean = jnp.dot(x, wg_ref[...], preferred_element_type=jnp.float32)
        raw = jnp.dot(x, wn_ref[...], preferred_element_type=jnp.float32)
        std = jnp.logaddexp(raw, 0.0) + NOISE_EPS
        cs[...] = clean
        ss[...] = std
        ns[...] = clean + z_ref[...] * std

    def phase_b(cs, ss, ns):
        clean = cs[...]
        std = ss[...]
        noisy = ns[...]

        work = noisy
        neg = jnp.float32(-jnp.inf)
        t8 = None
        t9 = None
        for it in range(K + 1):
            m = jnp.max(work, axis=-1, keepdims=True)
            if it == K - 1:
                t8 = m
            if it == K:
                t9 = m
            else:
                work = jnp.where(work == m, neg, work)

        mask = noisy > t9
        mx = jnp.max(noisy, axis=-1, keepdims=True)
        e = jnp.where(mask, jnp.exp(noisy - mx), 0.0)
        denom = jnp.sum(e, axis=-1, keepdims=True)
        gates_ref[...] = jnp.where(
            mask, (e / denom + 0.01) * (1.0 / 1.08), 0.0
        )

        kthresh = jnp.where(mask, t9, t8)
        arg = (clean - kthresh) / std * INV_SQRT2
        probs = 0.5 * (1.0 + lax.erf(arg))
        part = jnp.sum(probs.reshape(BLK // 8, 8, NUM_EXPERTS), axis=0)
        load_ref[...] = part.reshape(1, 8, NUM_EXPERTS)

    @pl.when(lax.rem(i, 2) == 0)
    def _():
        phase_a(c0, s0, n0)
        phase_b(c1, s1, n1)

    @pl.when(lax.rem(i, 2) == 1)
    def _():
        phase_a(c1, s1, n1)
        phase_b(c0, s0, n0)


@jax.jit
def _impl(x, wg_t, wn_t, z):
    sblk = lambda: pltpu.VMEM((BLK, NUM_EXPERTS), jnp.float32)
    gates, load_parts = pl.pallas_call(
        _router_kernel,
        grid=(NBLK + 1,),
        in_specs=[
            pl.BlockSpec((BLK, IN_DIM), lambda i: (jnp.minimum(i, NBLK - 1), 0)),
            pl.BlockSpec((IN_DIM, NUM_EXPERTS), lambda i: (0, 0)),
            pl.BlockSpec((IN_DIM, NUM_EXPERTS), lambda i: (0, 0)),
            pl.BlockSpec((BLK, NUM_EXPERTS), lambda i: (jnp.minimum(i, NBLK - 1), 0)),
        ],
        out_specs=[
            pl.BlockSpec((BLK, NUM_EXPERTS), lambda i: (jnp.maximum(i - 1, 0), 0)),
            pl.BlockSpec((1, 8, NUM_EXPERTS), lambda i: (jnp.maximum(i - 1, 0), 0, 0)),
        ],
        out_shape=[
            jax.ShapeDtypeStruct((N_TOKENS, NUM_EXPERTS), jnp.float32),
            jax.ShapeDtypeStruct((NBLK, 8, NUM_EXPERTS), jnp.float32),
        ],
        scratch_shapes=[sblk(), sblk(), sblk(), sblk(), sblk(), sblk()],
    )(x, wg_t, wn_t, z)
    return gates, jnp.sum(load_parts, axis=(0, 1))


def kernel(x, W_g, W_noise):
    return _impl(x, W_g.T, W_noise.T, _Z)


# speedup vs baseline: 1.8243x; 1.0582x over previous
import jax
import jax.numpy as jnp
from jax import lax
from jax.experimental import pallas as pl
from jax.experimental.pallas import tpu as pltpu

N_TOKENS = 8192
IN_DIM = 4096
NUM_EXPERTS = 64
K = 8
NOISE_EPS = 0.01
BLK = 1024
NBLK = N_TOKENS // BLK
INV_SQRT2 = 0.7071067811865476

_Z = jax.random.normal(
    jax.random.key(42), (N_TOKENS, NUM_EXPERTS), dtype=jnp.float32
)


def _router_kernel(
    x_ref, w_ref, z_ref, gates_ref, load_ref,
    c0, s0, n0, c1, s1, n1,
):
    i = pl.program_id(0)

    def phase_a(cs, ss, ns):
        x = x_ref[...]
        both = jnp.dot(x, w_ref[...], preferred_element_type=jnp.float32)
        clean = both[:, :NUM_EXPERTS]
        raw = both[:, NUM_EXPERTS:]
        std = jnp.logaddexp(raw, 0.0) + NOISE_EPS
        cs[...] = clean
        ss[...] = std
        ns[...] = clean + z_ref[...] * std

    def phase_b(cs, ss, ns):
        clean = cs[...]
        std = ss[...]
        noisy = ns[...]

        work = noisy
        neg = jnp.float32(-jnp.inf)
        t8 = None
        t9 = None
        for it in range(K + 1):
            m = jnp.max(work, axis=-1, keepdims=True)
            if it == K - 1:
                t8 = m
            if it == K:
                t9 = m
            else:
                work = jnp.where(work == m, neg, work)

        mask = noisy > t9
        mx = jnp.max(noisy, axis=-1, keepdims=True)
        e = jnp.where(mask, jnp.exp(noisy - mx), 0.0)
        denom = jnp.sum(e, axis=-1, keepdims=True)
        gates_ref[...] = jnp.where(
            mask, (e / denom + 0.01) * (1.0 / 1.08), 0.0
        )

        kthresh = jnp.where(mask, t9, t8)
        arg = (clean - kthresh) / std * INV_SQRT2
        probs = 0.5 * (1.0 + lax.erf(arg))
        part = jnp.sum(probs.reshape(BLK // 8, 8, NUM_EXPERTS), axis=0)
        load_ref[...] = part.reshape(1, 8, NUM_EXPERTS)

    @pl.when(lax.rem(i, 2) == 0)
    def _():
        phase_a(c0, s0, n0)
        phase_b(c1, s1, n1)

    @pl.when(lax.rem(i, 2) == 1)
    def _():
        phase_a(c1, s1, n1)
        phase_b(c0, s0, n0)


@jax.jit
def _impl(x, w_both, z):
    sblk = lambda: pltpu.VMEM((BLK, NUM_EXPERTS), jnp.float32)
    gates, load_parts = pl.pallas_call(
        _router_kernel,
        grid=(NBLK + 1,),
        in_specs=[
            pl.BlockSpec((BLK, IN_DIM), lambda i: (jnp.minimum(i, NBLK - 1), 0)),
            pl.BlockSpec((IN_DIM, 2 * NUM_EXPERTS), lambda i: (0, 0)),
            pl.BlockSpec((BLK, NUM_EXPERTS), lambda i: (jnp.minimum(i, NBLK - 1), 0)),
        ],
        out_specs=[
            pl.BlockSpec((BLK, NUM_EXPERTS), lambda i: (jnp.maximum(i - 1, 0), 0)),
            pl.BlockSpec((1, 8, NUM_EXPERTS), lambda i: (jnp.maximum(i - 1, 0), 0, 0)),
        ],
        out_shape=[
            jax.ShapeDtypeStruct((N_TOKENS, NUM_EXPERTS), jnp.float32),
            jax.ShapeDtypeStruct((NBLK, 8, NUM_EXPERTS), jnp.float32),
        ],
        scratch_shapes=[sblk(), sblk(), sblk(), sblk(), sblk(), sblk()],
    )(x, w_both, z)
    return gates, jnp.sum(load_parts, axis=(0, 1))


def kernel(x, W_g, W_noise):
    w_both = jnp.concatenate([W_g.T, W_noise.T], axis=1)
    return _impl(x, w_both, _Z)
